# baseline (speedup 1.0000x reference)
import functools

import jax
import jax.numpy as jnp
from jax import lax
from jax.experimental import pallas as pl
from jax.experimental.pallas import tpu as pltpu
from jax.experimental.pallas import tpu_sc as plsc

D4 = 256


def _sc_temporal(idx, table, batch):
    n_idx = idx.shape[0]

    mesh = plsc.VectorSubcoreMesh(core_axis_name="c", subcore_axis_name="s")

    @functools.partial(
        pl.kernel,
        mesh=mesh,
        out_type=jax.ShapeDtypeStruct((batch, 4 * D4), jnp.float32),
        scratch_types=[
            pltpu.VMEM((n_idx,), jnp.int32),
            pltpu.VMEM((n_idx, D4), jnp.float32),
            pltpu.VMEM((batch, D4), jnp.float32),
            pltpu.SemaphoreType.DMA,
        ],
    )
    def sc_kernel(idx_hbm, tbl_hbm, out_hbm, idx_v, rows_v, zero_v, sem):
        wid = lax.axis_index("s") * 2 + lax.axis_index("c")

        @pl.when(wid == 0)
        def _():
            pltpu.sync_copy(idx_hbm, idx_v)
            pltpu.async_copy(tbl_hbm.at[idx_v], rows_v, sem).wait()
            pltpu.sync_copy(rows_v.at[pl.ds(0, batch)],
                            out_hbm.at[:, pl.ds(0 * D4, D4)])
            pltpu.sync_copy(rows_v.at[pl.ds(batch, batch)],
                            out_hbm.at[:, pl.ds(1 * D4, D4)])
            pltpu.sync_copy(rows_v.at[pl.ds(2 * batch, batch)],
                            out_hbm.at[:, pl.ds(2 * D4, D4)])
            zvec = jnp.zeros((16,), jnp.float32)
            for j in range(batch):
                for i in range(D4 // 16):
                    zero_v[j, pl.ds(i * 16, 16)] = zvec
            pltpu.sync_copy(zero_v, out_hbm.at[:, pl.ds(3 * D4, D4)])

    return sc_kernel(idx, table)


import numpy as np


def _pe_tables(seq, d_model, seq_block):
    d = np.arange(d_model)
    div = np.exp((d // 2 * 2).astype(np.float64) * (-np.log(10000.0) / d_model))
    off = np.where(d % 2 == 1, np.pi / 2, 0.0)
    a = np.arange(8)[:, None] * div[None, :] + off[None, :]
    b = (8.0 * np.arange(seq_block // 8))[:, None] * div[None, :]
    dd = (float(seq_block) * np.arange(seq // seq_block))[:, None] * div[None, :]
    as_f32 = lambda v: jnp.asarray(v.astype(np.float32))
    return (as_f32(np.sin(a)), as_f32(np.cos(a)),
            as_f32(np.sin(b)), as_f32(np.cos(b)),
            as_f32(np.sin(dd)[:, None, :]), as_f32(np.cos(dd)[:, None, :]))


def _tc_body(x_ref, sa_ref, ca_ref, sb_ref, cb_ref, sd_ref, cd_ref, t_ref,
             o_ref):
    batch, seq_block, d_model = x_ref.shape
    sd = sd_ref[0]
    cd = cd_ref[0]
    sb = sb_ref[...]
    cb = cb_ref[...]
    sbd = sb * cd + cb * sd
    cbd = cb * cd - sb * sd
    pe_blk = (sa_ref[...][None, :, :] * cbd[:, None, :]
              + ca_ref[...][None, :, :] * sbd[:, None, :])
    pe_blk = pe_blk.reshape(seq_block, d_model)
    o_ref[...] = x_ref[...] + pe_blk[None, :, :] + t_ref[...][:, None, :]


def _tc_add(x, temporal, seq_block):
    batch, seq, d_model = x.shape
    sa, ca, sb, cb, sd, cd = _pe_tables(seq, d_model, seq_block)
    g = seq_block // 8
    grid = (seq // seq_block,)
    return pl.pallas_call(
        _tc_body,
        grid=grid,
        in_specs=[
            pl.BlockSpec((batch, seq_block, d_model), lambda i: (0, i, 0)),
            pl.BlockSpec((8, d_model), lambda i: (0, 0)),
            pl.BlockSpec((8, d_model), lambda i: (0, 0)),
            pl.BlockSpec((g, d_model), lambda i: (0, 0)),
            pl.BlockSpec((g, d_model), lambda i: (0, 0)),
            pl.BlockSpec((1, 1, d_model), lambda i: (i, 0, 0)),
            pl.BlockSpec((1, 1, d_model), lambda i: (i, 0, 0)),
            pl.BlockSpec((batch, d_model), lambda i: (0, 0)),
        ],
        out_specs=pl.BlockSpec((batch, seq_block, d_model), lambda i: (0, i, 0)),
        out_shape=jax.ShapeDtypeStruct((batch, seq, d_model), jnp.float32),
        compiler_params=pltpu.CompilerParams(
            dimension_semantics=("arbitrary",),
        ),
    )(x, sa, ca, sb, cb, sd, cd, temporal)


def kernel(x, hour, day_of_week, month, pe, hour_emb, dow_emb, month_emb):
    batch, seq, d_model = x.shape
    n_hour = hour_emb.shape[0]
    n_dow = dow_emb.shape[0]

    table = jnp.concatenate([hour_emb, dow_emb, month_emb], axis=0)
    idx = jnp.concatenate([
        hour.astype(jnp.int32),
        day_of_week.astype(jnp.int32) + n_hour,
        month.astype(jnp.int32) + n_hour + n_dow,
    ])
    n_idx = 16
    idx = jnp.pad(idx, (0, n_idx - idx.shape[0]))

    temporal = _sc_temporal(idx, table, batch)

    seq_block = 256
    return _tc_add(x, temporal, seq_block)


# speedup vs baseline: 1.0127x; 1.0127x over previous
import functools

import jax
import jax.numpy as jnp
from jax import lax
from jax.experimental import pallas as pl
from jax.experimental.pallas import tpu as pltpu
from jax.experimental.pallas import tpu_sc as plsc

D4 = 256


def _sc_temporal(idx, table, batch):
    n_idx = idx.shape[0]

    mesh = plsc.VectorSubcoreMesh(core_axis_name="c", subcore_axis_name="s")

    @functools.partial(
        pl.kernel,
        mesh=mesh,
        out_type=jax.ShapeDtypeStruct((batch, 4 * D4), jnp.float32),
        scratch_types=[
            pltpu.VMEM((n_idx,), jnp.int32),
            pltpu.VMEM((n_idx, D4), jnp.float32),
            pltpu.VMEM((batch, D4), jnp.float32),
            pltpu.SemaphoreType.DMA,
        ],
    )
    def sc_kernel(idx_hbm, tbl_hbm, out_hbm, idx_v, rows_v, zero_v, sem):
        wid = lax.axis_index("s") * 2 + lax.axis_index("c")

        @pl.when(wid == 0)
        def _():
            pltpu.sync_copy(idx_hbm, idx_v)
            pltpu.async_copy(tbl_hbm.at[idx_v], rows_v, sem).wait()
            pltpu.sync_copy(rows_v.at[pl.ds(0, batch)],
                            out_hbm.at[:, pl.ds(0 * D4, D4)])
            pltpu.sync_copy(rows_v.at[pl.ds(batch, batch)],
                            out_hbm.at[:, pl.ds(1 * D4, D4)])
            pltpu.sync_copy(rows_v.at[pl.ds(2 * batch, batch)],
                            out_hbm.at[:, pl.ds(2 * D4, D4)])
            zvec = jnp.zeros((16,), jnp.float32)
            for j in range(batch):
                for i in range(D4 // 16):
                    zero_v[j, pl.ds(i * 16, 16)] = zvec
            pltpu.sync_copy(zero_v, out_hbm.at[:, pl.ds(3 * D4, D4)])

    return sc_kernel(idx, table)


import numpy as np


def _pe_tables(seq, d_model, seq_block):
    d = np.arange(d_model)
    div = np.exp((d // 2 * 2).astype(np.float64) * (-np.log(10000.0) / d_model))
    off = np.where(d % 2 == 1, np.pi / 2, 0.0)
    a = np.arange(8)[:, None] * div[None, :] + off[None, :]
    b = (8.0 * np.arange(seq_block // 8))[:, None] * div[None, :]
    dd = (float(seq_block) * np.arange(seq // seq_block))[:, None] * div[None, :]
    as_f32 = lambda v: jnp.asarray(v.astype(np.float32))
    return (as_f32(np.sin(a)), as_f32(np.cos(a)),
            as_f32(np.sin(b)), as_f32(np.cos(b)),
            as_f32(np.sin(dd)[:, None, :]), as_f32(np.cos(dd)[:, None, :]))


def _tc_body(x_ref, sa_ref, ca_ref, sb_ref, cb_ref, sd_ref, cd_ref, t_ref,
             o_ref):
    batch, seq_block, d_model = x_ref.shape
    sd = sd_ref[0]
    cd = cd_ref[0]
    sb = sb_ref[...]
    cb = cb_ref[...]
    sbd = sb * cd + cb * sd
    cbd = cb * cd - sb * sd
    pe_blk = (sa_ref[...][None, :, :] * cbd[:, None, :]
              + ca_ref[...][None, :, :] * sbd[:, None, :])
    pe_blk = pe_blk.reshape(seq_block, d_model)
    o_ref[...] = x_ref[...] + pe_blk[None, :, :] + t_ref[...][:, None, :]


def _tc_add(x, temporal, seq_block):
    batch, seq, d_model = x.shape
    sa, ca, sb, cb, sd, cd = _pe_tables(seq, d_model, seq_block)
    g = seq_block // 8
    grid = (seq // seq_block,)
    return pl.pallas_call(
        _tc_body,
        grid=grid,
        in_specs=[
            pl.BlockSpec((batch, seq_block, d_model), lambda i: (0, i, 0)),
            pl.BlockSpec((8, d_model), lambda i: (0, 0)),
            pl.BlockSpec((8, d_model), lambda i: (0, 0)),
            pl.BlockSpec((g, d_model), lambda i: (0, 0)),
            pl.BlockSpec((g, d_model), lambda i: (0, 0)),
            pl.BlockSpec((1, 1, d_model), lambda i: (i, 0, 0)),
            pl.BlockSpec((1, 1, d_model), lambda i: (i, 0, 0)),
            pl.BlockSpec((batch, d_model), lambda i: (0, 0)),
        ],
        out_specs=pl.BlockSpec((batch, seq_block, d_model), lambda i: (0, i, 0)),
        out_shape=jax.ShapeDtypeStruct((batch, seq, d_model), jnp.float32),
        compiler_params=pltpu.CompilerParams(
            dimension_semantics=("arbitrary",),
        ),
    )(x, sa, ca, sb, cb, sd, cd, temporal)


def kernel(x, hour, day_of_week, month, pe, hour_emb, dow_emb, month_emb):
    batch, seq, d_model = x.shape
    n_hour = hour_emb.shape[0]
    n_dow = dow_emb.shape[0]

    table = jnp.concatenate([hour_emb, dow_emb, month_emb], axis=0)
    idx = jnp.concatenate([
        hour.astype(jnp.int32),
        day_of_week.astype(jnp.int32) + n_hour,
        month.astype(jnp.int32) + n_hour + n_dow,
    ])
    n_idx = 16
    idx = jnp.pad(idx, (0, n_idx - idx.shape[0]))

    temporal = _sc_temporal(idx, table, batch)

    seq_block = 512
    return _tc_add(x, temporal, seq_block)


# speedup vs baseline: 1.0142x; 1.0014x over previous
import functools

import jax
import jax.numpy as jnp
from jax import lax
from jax.experimental import pallas as pl
from jax.experimental.pallas import tpu as pltpu
from jax.experimental.pallas import tpu_sc as plsc

D4 = 256


def _sc_temporal(idx_flat, table, batch, n_pad):

    mesh = plsc.VectorSubcoreMesh(core_axis_name="c", subcore_axis_name="s")

    @functools.partial(
        pl.kernel,
        mesh=mesh,
        out_type=jax.ShapeDtypeStruct((batch, 4 * D4), jnp.float32),
        scratch_types=[
            pltpu.VMEM((n_pad,), jnp.int32),
            pltpu.VMEM((n_pad, D4), jnp.float32),
            pltpu.VMEM((batch, D4), jnp.float32),
            pltpu.SemaphoreType.DMA,
        ],
    )
    def sc_kernel(idx_hbm, tbl_hbm, out_hbm, idx_v, rows_v, zero_v, sem):
        wid = lax.axis_index("s") * 2 + lax.axis_index("c")

        for w in range(3):
            @pl.when(wid == w)
            def _(w=w):
                pltpu.sync_copy(idx_hbm.at[pl.ds(w * n_pad, n_pad)], idx_v)
                pltpu.async_copy(tbl_hbm.at[idx_v], rows_v, sem).wait()
                pltpu.sync_copy(rows_v.at[pl.ds(0, batch)],
                                out_hbm.at[:, pl.ds(w * D4, D4)])

        @pl.when(wid == 3)
        def _():
            zvec = jnp.zeros((16,), jnp.float32)
            for j in range(batch):
                for i in range(D4 // 16):
                    zero_v[j, pl.ds(i * 16, 16)] = zvec
            pltpu.sync_copy(zero_v, out_hbm.at[:, pl.ds(3 * D4, D4)])

    return sc_kernel(idx_flat, table)


import numpy as np


def _pe_tables(seq, d_model, seq_block):
    d = np.arange(d_model)
    div = np.exp((d // 2 * 2).astype(np.float64) * (-np.log(10000.0) / d_model))
    off = np.where(d % 2 == 1, np.pi / 2, 0.0)
    a = np.arange(8)[:, None] * div[None, :] + off[None, :]
    b = (8.0 * np.arange(seq_block // 8))[:, None] * div[None, :]
    dd = (float(seq_block) * np.arange(seq // seq_block))[:, None] * div[None, :]
    as_f32 = lambda v: jnp.asarray(v.astype(np.float32))
    return (as_f32(np.sin(a)), as_f32(np.cos(a)),
            as_f32(np.sin(b)), as_f32(np.cos(b)),
            as_f32(np.sin(dd)[:, None, :]), as_f32(np.cos(dd)[:, None, :]))


def _tc_body(x_ref, sa_ref, ca_ref, sb_ref, cb_ref, sd_ref, cd_ref, t_ref,
             o_ref):
    batch, seq_block, d_model = x_ref.shape
    sd = sd_ref[0]
    cd = cd_ref[0]
    sb = sb_ref[...]
    cb = cb_ref[...]
    sbd = sb * cd + cb * sd
    cbd = cb * cd - sb * sd
    pe_blk = (sa_ref[...][None, :, :] * cbd[:, None, :]
              + ca_ref[...][None, :, :] * sbd[:, None, :])
    pe_blk = pe_blk.reshape(seq_block, d_model)
    o_ref[...] = x_ref[...] + pe_blk[None, :, :] + t_ref[...][:, None, :]


def _tc_add(x, temporal, seq_block):
    batch, seq, d_model = x.shape
    sa, ca, sb, cb, sd, cd = _pe_tables(seq, d_model, seq_block)
    g = seq_block // 8
    grid = (seq // seq_block,)
    return pl.pallas_call(
        _tc_body,
        grid=grid,
        in_specs=[
            pl.BlockSpec((batch, seq_block, d_model), lambda i: (0, i, 0)),
            pl.BlockSpec((8, d_model), lambda i: (0, 0)),
            pl.BlockSpec((8, d_model), lambda i: (0, 0)),
            pl.BlockSpec((g, d_model), lambda i: (0, 0)),
            pl.BlockSpec((g, d_model), lambda i: (0, 0)),
            pl.BlockSpec((1, 1, d_model), lambda i: (i, 0, 0)),
            pl.BlockSpec((1, 1, d_model), lambda i: (i, 0, 0)),
            pl.BlockSpec((batch, d_model), lambda i: (0, 0)),
        ],
        out_specs=pl.BlockSpec((batch, seq_block, d_model), lambda i: (0, i, 0)),
        out_shape=jax.ShapeDtypeStruct((batch, seq, d_model), jnp.float32),
        compiler_params=pltpu.CompilerParams(
            dimension_semantics=("arbitrary",),
        ),
    )(x, sa, ca, sb, cb, sd, cd, temporal)


def kernel(x, hour, day_of_week, month, pe, hour_emb, dow_emb, month_emb):
    batch, seq, d_model = x.shape
    n_hour = hour_emb.shape[0]
    n_dow = dow_emb.shape[0]

    table = jnp.concatenate([hour_emb, dow_emb, month_emb], axis=0)
    pad = (0, 8 - batch)
    idx_flat = jnp.concatenate([
        jnp.pad(hour.astype(jnp.int32), pad),
        jnp.pad(day_of_week.astype(jnp.int32) + n_hour, pad),
        jnp.pad(month.astype(jnp.int32) + n_hour + n_dow, pad),
    ])

    temporal = _sc_temporal(idx_flat, table, batch, 8)

    seq_block = 512
    return _tc_add(x, temporal, seq_block)
